# Initial kernel scaffold; baseline (speedup 1.0000x reference)
import jax
import jax.numpy as jnp
from jax.experimental import pallas as pl


def kernel(x, gate_w, w_gate, w_up, w_down, sw_gate, sw_up, sw_down):
    raise NotImplementedError("write your pallas kernel here")


# speedup vs baseline: 1.6391x; 1.6391x over previous
import functools

import jax
import jax.numpy as jnp
from jax.experimental import pallas as pl
from jax.experimental.pallas import tpu as pltpu

E = 8
TOP_K = 2
D = 1024
FF = 1024
T = 2048

TT = 8
TB = T // TT


def _moe_body(x_ref, rnd_ref, wg_ref, wu_ref, wd_ref,
              swg_ref, swu_ref, swd_ref, out_ref,
              wgb, wub, wdb):
    i = pl.program_id(0)
    tt = pl.program_id(1)

    @pl.when(jnp.logical_and(tt == 0, i < E))
    def _():
        wgb[...] = wg_ref[0].astype(jnp.bfloat16)
        wub[...] = wu_ref[0].astype(jnp.bfloat16)
        wdb[...] = wd_ref[0].astype(jnp.bfloat16)

    @pl.when(jnp.logical_and(tt == 0, i == E))
    def _():
        wgb[...] = swg_ref[...].astype(jnp.bfloat16)
        wub[...] = swu_ref[...].astype(jnp.bfloat16)
        wdb[...] = swd_ref[...].astype(jnp.bfloat16)

    xt = x_ref[...].astype(jnp.bfloat16)
    a = jnp.dot(xt, wgb[...], preferred_element_type=jnp.float32)
    b = jnp.dot(xt, wub[...], preferred_element_type=jnp.float32)
    h = (a * jax.lax.logistic(a) * b).astype(jnp.bfloat16)
    o = jnp.dot(h, wdb[...], preferred_element_type=jnp.float32)

    r = rnd_ref[...]
    m = jnp.max(r, axis=1, keepdims=True)
    ex = jnp.exp(r - m)
    s = ex / jnp.sum(ex, axis=1, keepdims=True)
    m1 = jnp.max(s, axis=1, keepdims=True)
    is1 = s == m1
    m2 = jnp.max(jnp.where(is1, -jnp.inf, s), axis=1, keepdims=True)
    is2 = jnp.logical_and(s == m2, jnp.logical_not(is1))
    wt = s * jnp.logical_or(is1, is2).astype(s.dtype)
    lane = jax.lax.broadcasted_iota(jnp.int32, (TB, E), 1)
    col = jnp.sum(jnp.where(lane == i, wt, 0.0), axis=1, keepdims=True)
    col = jnp.where(i == E, 1.0, col)

    contrib = o * col
    rows = pl.ds(tt * TB, TB)

    @pl.when(i == 0)
    def _():
        out_ref[rows, :] = contrib

    @pl.when(i > 0)
    def _():
        out_ref[rows, :] += contrib


@jax.jit
def kernel(x, gate_w, w_gate, w_up, w_down, sw_gate, sw_up, sw_down):
    del gate_w
    rnd = jax.random.normal(jax.random.key(42), (T, E), dtype=jnp.float32)

    grid = (E + 1, TT)
    out = pl.pallas_call(
        _moe_body,
        grid=grid,
        in_specs=[
            pl.BlockSpec((TB, D), lambda i, tt: (tt, 0)),
            pl.BlockSpec((TB, E), lambda i, tt: (tt, 0)),
            pl.BlockSpec((1, D, FF), lambda i, tt: (jnp.minimum(i, E - 1), 0, 0)),
            pl.BlockSpec((1, D, FF), lambda i, tt: (jnp.minimum(i, E - 1), 0, 0)),
            pl.BlockSpec((1, FF, D), lambda i, tt: (jnp.minimum(i, E - 1), 0, 0)),
            pl.BlockSpec((D, FF), lambda i, tt: (0, 0)),
            pl.BlockSpec((D, FF), lambda i, tt: (0, 0)),
            pl.BlockSpec((FF, D), lambda i, tt: (0, 0)),
        ],
        out_specs=pl.BlockSpec((T, D), lambda i, tt: (0, 0)),
        out_shape=jax.ShapeDtypeStruct((T, D), jnp.float32),
        scratch_shapes=[
            pltpu.VMEM((D, FF), jnp.bfloat16),
            pltpu.VMEM((D, FF), jnp.bfloat16),
            pltpu.VMEM((FF, D), jnp.bfloat16),
        ],
    )(x, rnd, w_gate, w_up, w_down, sw_gate, sw_up, sw_down)
    return out
